# Initial kernel scaffold; baseline (speedup 1.0000x reference)
import functools

import jax
import jax.numpy as jnp
from jax import lax
from jax.experimental import pallas as pl
from jax.experimental.pallas import tpu as pltpu
from jax.experimental.pallas import tpu_sc as plsc

N = 10000
NP = 10240
E = 320000
CH = 128
NW = 32
CPW = 79
EP = NW * CPW * CH
DUMMY = N + 200
RPT = NP // 16

_mesh = plsc.VectorSubcoreMesh(core_axis_name="c", subcore_axis_name="s")


@functools.partial(
    pl.kernel,
    out_type=jax.ShapeDtypeStruct((2 * NP, 16), jnp.float32),
    mesh=_mesh,
    scratch_types=[
        pltpu.VMEM((CH,), jnp.int32),
        pltpu.VMEM((CH, 16), jnp.float32),
        pltpu.VMEM_SHARED((NP, 16), jnp.float32),
    ],
)
def _sc_degree(dst_hbm, ones_hbm, zeros_hbm, out_hbm, dst_v, ones_v, acc_sh):
    cid = lax.axis_index("c")
    sid = lax.axis_index("s")
    wid = sid * 2 + cid
    pltpu.sync_copy(ones_hbm, ones_v)
    pltpu.sync_copy(zeros_hbm, acc_sh.at[pl.ds(sid * RPT, RPT)])
    plsc.subcore_barrier()

    def body(i, carry):
        base = (i * NW + wid) * CH
        pltpu.sync_copy(dst_hbm.at[pl.ds(base, CH)], dst_v)
        pltpu.sync_copy(ones_v, acc_sh.at[dst_v], add=True)
        return carry

    lax.fori_loop(0, CPW, body, 0)
    plsc.subcore_barrier()
    pltpu.sync_copy(
        acc_sh.at[pl.ds(sid * RPT, RPT)],
        out_hbm.at[pl.ds(cid * NP + sid * RPT, RPT)],
    )


@functools.partial(
    pl.kernel,
    out_type=jax.ShapeDtypeStruct((2 * NP, 128), jnp.float32),
    mesh=_mesh,
    scratch_types=[
        pltpu.VMEM((CH,), jnp.int32),
        pltpu.VMEM((CH,), jnp.int32),
        pltpu.VMEM((CH, 128), jnp.float32),
        pltpu.VMEM_SHARED((NP, 128), jnp.float32),
        pltpu.SemaphoreType.DMA,
    ],
)
def _sc_scatter(z_hbm, src_hbm, dst_hbm, zeros_hbm, out_hbm,
                src_v, dst_v, rows_v, acc_sh, sem):
    cid = lax.axis_index("c")
    sid = lax.axis_index("s")
    wid = sid * 2 + cid
    pltpu.sync_copy(zeros_hbm, acc_sh.at[pl.ds(sid * RPT, RPT)])
    plsc.subcore_barrier()

    def body(i, carry):
        base = (i * NW + wid) * CH
        pltpu.sync_copy(src_hbm.at[pl.ds(base, CH)], src_v)
        pltpu.sync_copy(dst_hbm.at[pl.ds(base, CH)], dst_v)
        pltpu.async_copy(z_hbm.at[src_v], rows_v, sem).wait()
        pltpu.sync_copy(rows_v, acc_sh.at[dst_v], add=True)
        return carry

    lax.fori_loop(0, CPW, body, 0)
    plsc.subcore_barrier()
    pltpu.sync_copy(
        acc_sh.at[pl.ds(sid * RPT, RPT)],
        out_hbm.at[pl.ds(cid * NP + sid * RPT, RPT)],
    )


BR = 256


def _tc_a_body(x_ref, w_ref, da_ref, db_ref, o_ref):
    deg = da_ref[:, 0:1] + db_ref[:, 0:1] + 1.0
    dis = lax.rsqrt(deg)
    xw = jnp.dot(x_ref[...], w_ref[...], preferred_element_type=jnp.float32,
                 precision=lax.Precision.HIGHEST)
    o_ref[...] = xw * dis


def _tc_b_body(aa_ref, ab_ref, z_ref, da_ref, db_ref, b_ref, w_ref, o_ref):
    i = pl.program_id(0)
    deg = da_ref[:, 0:1] + db_ref[:, 0:1] + 1.0
    dis = lax.rsqrt(deg)
    s = (aa_ref[...] + ab_ref[...] + z_ref[...]) * dis + b_ref[...]
    h = jnp.maximum(s, 0.0)
    row = lax.broadcasted_iota(jnp.int32, (BR, 1), 0) + i * BR
    h = jnp.where(row < N, h, 0.0)
    z2 = jnp.dot(h, w_ref[...], preferred_element_type=jnp.float32,
                 precision=lax.Precision.HIGHEST)
    o_ref[...] = z2 * dis


BRC = 400


def _tc_c_body(aa_ref, ab_ref, z_ref, da_ref, db_ref, b_ref, o_ref):
    deg = da_ref[:, 0:1] + db_ref[:, 0:1] + 1.0
    dis = lax.rsqrt(deg)
    o_ref[...] = (aa_ref[...] + ab_ref[...] + z_ref[...]) * dis + b_ref[...]


def _tc_a(x_p, W, dA, dB):
    return pl.pallas_call(
        _tc_a_body,
        grid=(NP // BR,),
        in_specs=[
            pl.BlockSpec((BR, 128), lambda i: (i, 0)),
            pl.BlockSpec((128, 128), lambda i: (0, 0)),
            pl.BlockSpec((BR, 16), lambda i: (i, 0)),
            pl.BlockSpec((BR, 16), lambda i: (i, 0)),
        ],
        out_specs=pl.BlockSpec((BR, 128), lambda i: (i, 0)),
        out_shape=jax.ShapeDtypeStruct((NP, 128), jnp.float32),
    )(x_p, W, dA, dB)


def _tc_b(accA, accB, z1, dA, dB, b1, W2):
    return pl.pallas_call(
        _tc_b_body,
        grid=(NP // BR,),
        in_specs=[
            pl.BlockSpec((BR, 128), lambda i: (i, 0)),
            pl.BlockSpec((BR, 128), lambda i: (i, 0)),
            pl.BlockSpec((BR, 128), lambda i: (i, 0)),
            pl.BlockSpec((BR, 16), lambda i: (i, 0)),
            pl.BlockSpec((BR, 16), lambda i: (i, 0)),
            pl.BlockSpec((128,), lambda i: (0,)),
            pl.BlockSpec((128, 128), lambda i: (0, 0)),
        ],
        out_specs=pl.BlockSpec((BR, 128), lambda i: (i, 0)),
        out_shape=jax.ShapeDtypeStruct((NP, 128), jnp.float32),
    )(accA, accB, z1, dA, dB, b1, W2)


def _tc_c(accA, accB, z2, dA, dB, b2):
    return pl.pallas_call(
        _tc_c_body,
        grid=(N // BRC,),
        in_specs=[
            pl.BlockSpec((BRC, 128), lambda i: (i, 0)),
            pl.BlockSpec((BRC, 128), lambda i: (i, 0)),
            pl.BlockSpec((BRC, 128), lambda i: (i, 0)),
            pl.BlockSpec((BRC, 16), lambda i: (i, 0)),
            pl.BlockSpec((BRC, 16), lambda i: (i, 0)),
            pl.BlockSpec((128,), lambda i: (0,)),
        ],
        out_specs=pl.BlockSpec((BRC, 128), lambda i: (i, 0)),
        out_shape=jax.ShapeDtypeStruct((N, 128), jnp.float32),
    )(accA, accB, z2, dA, dB, b2)


def kernel(x, edge_index, W1, b1, W2, b2):
    src = edge_index[0].astype(jnp.int32)
    dst = edge_index[1].astype(jnp.int32)
    pad = jnp.full((EP - E,), DUMMY, jnp.int32)
    src_p = jnp.concatenate([src, pad])
    dst_p = jnp.concatenate([dst, pad])
    x_p = jnp.zeros((NP, 128), jnp.float32).at[:N].set(x)
    ones16 = jnp.ones((CH, 16), jnp.float32)
    zeros16 = jnp.zeros((RPT, 16), jnp.float32)
    zeros128 = jnp.zeros((RPT, 128), jnp.float32)

    degout = _sc_degree(dst_p, ones16, zeros16)
    dA, dB = degout[:NP], degout[NP:]

    z1 = _tc_a(x_p, W1, dA, dB)
    acc1 = _sc_scatter(z1, src_p, dst_p, zeros128)
    z2 = _tc_b(acc1[:NP], acc1[NP:], z1, dA, dB, b1, W2)
    acc2 = _sc_scatter(z2, src_p, dst_p, zeros128)
    return _tc_c(acc2[:NP], acc2[NP:], z2, dA, dB, b2)


# speedup vs baseline: 10.1495x; 10.1495x over previous
import functools

import jax
import jax.numpy as jnp
from jax import lax
from jax.experimental import pallas as pl
from jax.experimental.pallas import tpu as pltpu
from jax.experimental.pallas import tpu_sc as plsc

N = 10000
NP = 10240
E = 320000
CH = 128
NW = 32
CPW = 79
EP = NW * CPW * CH
DUMMY = N + 200
RPT = NP // 16

_mesh = plsc.VectorSubcoreMesh(core_axis_name="c", subcore_axis_name="s")


@functools.partial(
    pl.kernel,
    out_type=jax.ShapeDtypeStruct((2 * NP, 128), jnp.float32),
    mesh=_mesh,
    scratch_types=[
        pltpu.VMEM((CH,), jnp.int32),
        pltpu.VMEM((CH, 128), jnp.float32),
        pltpu.VMEM_SHARED((NP, 128), jnp.float32),
    ],
)
def _sc_degree(dst_hbm, ones_hbm, zeros_hbm, out_hbm, dst_v, ones_v, acc_sh):
    cid = lax.axis_index("c")
    sid = lax.axis_index("s")
    wid = sid * 2 + cid
    pltpu.sync_copy(ones_hbm, ones_v)
    pltpu.sync_copy(zeros_hbm, acc_sh.at[pl.ds(sid * RPT, RPT)])
    plsc.subcore_barrier()

    def body(i, carry):
        base = (i * NW + wid) * CH
        pltpu.sync_copy(dst_hbm.at[pl.ds(base, CH)], dst_v)
        pltpu.sync_copy(ones_v, acc_sh.at[dst_v], add=True)
        return carry

    lax.fori_loop(0, CPW, body, 0)
    plsc.subcore_barrier()
    pltpu.sync_copy(
        acc_sh.at[pl.ds(sid * RPT, RPT)],
        out_hbm.at[pl.ds(cid * NP + sid * RPT, RPT)],
    )


@functools.partial(
    pl.kernel,
    out_type=jax.ShapeDtypeStruct((2 * NP, 128), jnp.float32),
    mesh=_mesh,
    scratch_types=[
        pltpu.VMEM((CH,), jnp.int32),
        pltpu.VMEM((CH,), jnp.int32),
        pltpu.VMEM((CH, 128), jnp.float32),
        pltpu.VMEM_SHARED((NP, 128), jnp.float32),
        pltpu.SemaphoreType.DMA,
    ],
)
def _sc_scatter(z_hbm, src_hbm, dst_hbm, zeros_hbm, out_hbm,
                src_v, dst_v, rows_v, acc_sh, sem):
    cid = lax.axis_index("c")
    sid = lax.axis_index("s")
    wid = sid * 2 + cid
    pltpu.sync_copy(zeros_hbm, acc_sh.at[pl.ds(sid * RPT, RPT)])
    plsc.subcore_barrier()

    def body(i, carry):
        base = (i * NW + wid) * CH
        pltpu.sync_copy(src_hbm.at[pl.ds(base, CH)], src_v)
        pltpu.sync_copy(dst_hbm.at[pl.ds(base, CH)], dst_v)
        pltpu.async_copy(z_hbm.at[src_v], rows_v, sem).wait()
        pltpu.sync_copy(rows_v, acc_sh.at[dst_v], add=True)
        return carry

    lax.fori_loop(0, CPW, body, 0)
    plsc.subcore_barrier()
    pltpu.sync_copy(
        acc_sh.at[pl.ds(sid * RPT, RPT)],
        out_hbm.at[pl.ds(cid * NP + sid * RPT, RPT)],
    )


BR = 256


def _tc_a_body(x_ref, w_ref, da_ref, db_ref, o_ref):
    deg = da_ref[:, 0:1] + db_ref[:, 0:1] + 1.0
    dis = lax.rsqrt(deg)
    xw = jnp.dot(x_ref[...], w_ref[...], preferred_element_type=jnp.float32,
                 precision=lax.Precision.HIGHEST)
    o_ref[...] = xw * dis


def _tc_b_body(aa_ref, ab_ref, z_ref, da_ref, db_ref, b_ref, w_ref, o_ref):
    i = pl.program_id(0)
    deg = da_ref[:, 0:1] + db_ref[:, 0:1] + 1.0
    dis = lax.rsqrt(deg)
    s = (aa_ref[...] + ab_ref[...] + z_ref[...]) * dis + b_ref[...]
    h = jnp.maximum(s, 0.0)
    row = lax.broadcasted_iota(jnp.int32, (BR, 1), 0) + i * BR
    h = jnp.where(row < N, h, 0.0)
    z2 = jnp.dot(h, w_ref[...], preferred_element_type=jnp.float32,
                 precision=lax.Precision.HIGHEST)
    o_ref[...] = z2 * dis


BRC = 400


def _tc_c_body(aa_ref, ab_ref, z_ref, da_ref, db_ref, b_ref, o_ref):
    deg = da_ref[:, 0:1] + db_ref[:, 0:1] + 1.0
    dis = lax.rsqrt(deg)
    o_ref[...] = (aa_ref[...] + ab_ref[...] + z_ref[...]) * dis + b_ref[...]


def _tc_a(x_p, W, dA, dB):
    return pl.pallas_call(
        _tc_a_body,
        grid=(NP // BR,),
        in_specs=[
            pl.BlockSpec((BR, 128), lambda i: (i, 0)),
            pl.BlockSpec((128, 128), lambda i: (0, 0)),
            pl.BlockSpec((BR, 128), lambda i: (i, 0)),
            pl.BlockSpec((BR, 128), lambda i: (i, 0)),
        ],
        out_specs=pl.BlockSpec((BR, 128), lambda i: (i, 0)),
        out_shape=jax.ShapeDtypeStruct((NP, 128), jnp.float32),
    )(x_p, W, dA, dB)


def _tc_b(accA, accB, z1, dA, dB, b1, W2):
    return pl.pallas_call(
        _tc_b_body,
        grid=(NP // BR,),
        in_specs=[
            pl.BlockSpec((BR, 128), lambda i: (i, 0)),
            pl.BlockSpec((BR, 128), lambda i: (i, 0)),
            pl.BlockSpec((BR, 128), lambda i: (i, 0)),
            pl.BlockSpec((BR, 128), lambda i: (i, 0)),
            pl.BlockSpec((BR, 128), lambda i: (i, 0)),
            pl.BlockSpec((128,), lambda i: (0,)),
            pl.BlockSpec((128, 128), lambda i: (0, 0)),
        ],
        out_specs=pl.BlockSpec((BR, 128), lambda i: (i, 0)),
        out_shape=jax.ShapeDtypeStruct((NP, 128), jnp.float32),
    )(accA, accB, z1, dA, dB, b1, W2)


def _tc_c(accA, accB, z2, dA, dB, b2):
    return pl.pallas_call(
        _tc_c_body,
        grid=(N // BRC,),
        in_specs=[
            pl.BlockSpec((BRC, 128), lambda i: (i, 0)),
            pl.BlockSpec((BRC, 128), lambda i: (i, 0)),
            pl.BlockSpec((BRC, 128), lambda i: (i, 0)),
            pl.BlockSpec((BRC, 128), lambda i: (i, 0)),
            pl.BlockSpec((BRC, 128), lambda i: (i, 0)),
            pl.BlockSpec((128,), lambda i: (0,)),
        ],
        out_specs=pl.BlockSpec((BRC, 128), lambda i: (i, 0)),
        out_shape=jax.ShapeDtypeStruct((N, 128), jnp.float32),
    )(accA, accB, z2, dA, dB, b2)


def kernel(x, edge_index, W1, b1, W2, b2):
    src = edge_index[0].astype(jnp.int32)
    dst = edge_index[1].astype(jnp.int32)
    pad = jnp.full((EP - E,), DUMMY, jnp.int32)
    src_p = jnp.concatenate([src, pad])
    dst_p = jnp.concatenate([dst, pad])
    x_p = jnp.zeros((NP, 128), jnp.float32).at[:N].set(x)
    ones128 = jnp.ones((CH, 128), jnp.float32)
    zeros128 = jnp.zeros((RPT, 128), jnp.float32)

    degout = _sc_degree(dst_p, ones128, zeros128)
    dA, dB = degout[:NP], degout[NP:]

    z1 = _tc_a(x_p, W1, dA, dB)
    acc1 = _sc_scatter(z1, src_p, dst_p, zeros128)
    z2 = _tc_b(acc1[:NP], acc1[NP:], z1, dA, dB, b1, W2)
    acc2 = _sc_scatter(z2, src_p, dst_p, zeros128)
    return _tc_c(acc2[:NP], acc2[NP:], z2, dA, dB, b2)
